# baseline (speedup 1.0000x reference)
import functools

import jax
import jax.numpy as jnp
import numpy as np
from jax.experimental import pallas as pl
from jax.experimental.pallas import tpu as pltpu

VOCAB = 30522
TYPES = 16
D = 768
H = 12
DH = 64
L = 2048
G = 128
R = 64
MAXD = 12
FF = 3072
RELVOC = 2 * MAXD + 1
TOTREL = RELVOC + 3
NB = L // R
T = L + G
TM = 136
GROWS = 16
SCALE = 1.0 / np.sqrt(DH)
NEG = -1e9

_pidx = np.arange(R)
_qidx = np.arange(3 * R)
_dloc = _qidx[None, :] - (R + _pidx[:, None])


def _np_rel_id(d):
    return np.where(d >= 0, np.minimum(d, MAXD), MAXD + np.minimum(-d, MAXD))


_L2L_IDS = _np_rel_id(_dloc)
_dg = np.arange(G)[None, :] - np.arange(G)[:, None]
_G2G_IDS = _np_rel_id(_dg)


def _gather_body(ids_ref, *refs):
    del ids_ref
    out_ref = refs[GROWS]
    for j in range(GROWS):
        out_ref[j, :] = refs[j][0, 0, :]


def _emb_gather(emb, ids):
    emb3 = emb.reshape(VOCAB, 1, D)

    def mk(j):
        return pl.BlockSpec((1, 1, D),
                            lambda i, ids_ref, j=j: (ids_ref[i * GROWS + j], 0, 0))

    grid_spec = pltpu.PrefetchScalarGridSpec(
        num_scalar_prefetch=1,
        grid=(T // GROWS,),
        in_specs=[mk(j) for j in range(GROWS)],
        out_specs=pl.BlockSpec((GROWS, D), lambda i, ids_ref: (i, 0)),
    )
    return pl.pallas_call(
        _gather_body,
        grid_spec=grid_spec,
        out_shape=jax.ShapeDtypeStruct((T, D), jnp.float32),
    )(ids, *([emb3] * GROWS))


def _embln_body(x_ref, tids_ref, temb_ref, s_ref, b_ref, o_ref):
    x = x_ref[...]
    tids = tids_ref[0, 0, :]
    oh = (tids[:, None] == jax.lax.broadcasted_iota(jnp.int32, (1, TYPES), 1)
          ).astype(jnp.float32)
    x = x + jnp.dot(oh, temb_ref[...], preferred_element_type=jnp.float32)
    m = jnp.mean(x, axis=-1, keepdims=True)
    v = jnp.mean(jnp.square(x - m), axis=-1, keepdims=True)
    o_ref[...] = (x - m) / jnp.sqrt(v + 1e-6) * s_ref[...] + b_ref[...]


def _embed_ln(rows, type_ids, type_emb, s, b):
    return pl.pallas_call(
        _embln_body,
        grid=(T // TM,),
        in_specs=[
            pl.BlockSpec((TM, D), lambda i: (i, 0)),
            pl.BlockSpec((1, 1, TM), lambda i: (i, 0, 0)),
            pl.BlockSpec((TYPES, D), lambda i: (0, 0)),
            pl.BlockSpec((1, D), lambda i: (0, 0)),
            pl.BlockSpec((1, D), lambda i: (0, 0)),
        ],
        out_specs=pl.BlockSpec((TM, D), lambda i: (i, 0)),
        out_shape=jax.ShapeDtypeStruct((T, D), jnp.float32),
    )(rows, type_ids.reshape(T // TM, 1, TM), type_emb,
      s.reshape(1, D), b.reshape(1, D))


def _mm_body(x_ref, w_ref, b_ref, o_ref, *, act, lowp=False):
    x = x_ref[...]
    w = w_ref[...]
    if lowp:
        x = x.astype(jnp.bfloat16)
        w = w.astype(jnp.bfloat16)
    y = jnp.dot(x, w, preferred_element_type=jnp.float32)
    y = y + b_ref[...]
    if act == 'gelu':
        y = jax.nn.gelu(y)
    elif act == 'sigmoid':
        y = jax.nn.sigmoid(y)
    o_ref[...] = y


def _matmul(x, w, b, act='none', lowp=False):
    M, K = x.shape
    N = w.shape[1]
    return pl.pallas_call(
        functools.partial(_mm_body, act=act, lowp=lowp),
        grid=(M // TM,),
        in_specs=[
            pl.BlockSpec((TM, K), lambda i: (i, 0)),
            pl.BlockSpec((K, N), lambda i: (0, 0)),
            pl.BlockSpec((1, N), lambda i: (0, 0)),
        ],
        out_specs=pl.BlockSpec((TM, N), lambda i: (i, 0)),
        out_shape=jax.ShapeDtypeStruct((M, N), jnp.float32),
    )(x, w, b.reshape(1, N))


def _mm_ln_body(x_ref, w_ref, b_ref, r_ref, s_ref, bb_ref, o_ref, *, lowp=False):
    x = x_ref[...]
    w = w_ref[...]
    if lowp:
        x = x.astype(jnp.bfloat16)
        w = w.astype(jnp.bfloat16)
    y = r_ref[...] + jnp.dot(x, w, preferred_element_type=jnp.float32) + b_ref[...]
    m = jnp.mean(y, axis=-1, keepdims=True)
    v = jnp.mean(jnp.square(y - m), axis=-1, keepdims=True)
    o_ref[...] = (y - m) / jnp.sqrt(v + 1e-6) * s_ref[...] + bb_ref[...]


def _matmul_ln(x, w, b, res, s, bb, lowp=False):
    M, K = x.shape
    N = w.shape[1]
    return pl.pallas_call(
        functools.partial(_mm_ln_body, lowp=lowp),
        grid=(M // TM,),
        in_specs=[
            pl.BlockSpec((TM, K), lambda i: (i, 0)),
            pl.BlockSpec((K, N), lambda i: (0, 0)),
            pl.BlockSpec((1, N), lambda i: (0, 0)),
            pl.BlockSpec((TM, N), lambda i: (i, 0)),
            pl.BlockSpec((1, N), lambda i: (0, 0)),
            pl.BlockSpec((1, N), lambda i: (0, 0)),
        ],
        out_specs=pl.BlockSpec((TM, N), lambda i: (i, 0)),
        out_shape=jax.ShapeDtypeStruct((M, N), jnp.float32),
    )(x, w, b.reshape(1, N), res, s.reshape(1, N), bb.reshape(1, N))


def _head_body(x_ref, w_ref, b_ref, o_ref):
    i = pl.program_id(0)
    y = jnp.dot(x_ref[...], w_ref[...], preferred_element_type=jnp.float32)
    y = y + b_ref[...]
    rows = i * TM + jax.lax.broadcasted_iota(jnp.int32, (TM, 1), 0)
    o_ref[...] = jax.nn.sigmoid(jnp.where(rows >= L, y[:, 1:2], y[:, 0:1]))


def _head(x, w2, b2):
    return pl.pallas_call(
        _head_body,
        grid=(T // TM,),
        in_specs=[
            pl.BlockSpec((TM, D), lambda i: (i, 0)),
            pl.BlockSpec((D, 2), lambda i: (0, 0)),
            pl.BlockSpec((1, 2), lambda i: (0, 0)),
        ],
        out_specs=pl.BlockSpec((TM, 1), lambda i: (i, 0)),
        out_shape=jax.ShapeDtypeStruct((T, 1), jnp.float32),
    )(x, w2, b2)


def _suffix_cumsum(bm):
    P, Q = bm.shape
    jq = jax.lax.broadcasted_iota(jnp.int32, (Q, Q), 0)
    iq = jax.lax.broadcasted_iota(jnp.int32, (Q, Q), 1)
    tq = (jq >= iq).astype(jnp.float32)
    w = jnp.dot(bm, tq, preferred_element_type=jnp.float32)
    rowtot = w[:, 0:1]
    jp = jax.lax.broadcasted_iota(jnp.int32, (P, P), 0)
    ip = jax.lax.broadcasted_iota(jnp.int32, (P, P), 1)
    tp = (jp > ip).astype(jnp.float32)
    tail = jnp.dot(rowtot.reshape(1, P), tp,
                   preferred_element_type=jnp.float32)
    return w + tail.reshape(P, 1)


def _mask_body(lbp_ref, gbp_ref, gbpc_ref, pid_ref,
               locm_ref, l2g_ref, g2g_ref, ind_ref, same_ref):
    segb = _suffix_cumsum(lbp_ref[...].astype(jnp.float32))
    gbpf = gbp_ref[...].astype(jnp.float32)
    jq = jax.lax.broadcasted_iota(jnp.int32, (G, G), 0)
    iq = jax.lax.broadcasted_iota(jnp.int32, (G, G), 1)
    tq = (jq >= iq).astype(jnp.float32)
    gseg_row = jnp.dot(gbpf, tq, preferred_element_type=jnp.float32)
    tq2 = (iq >= jq).astype(jnp.float32)
    gseg_col = jnp.dot(tq2, gbpc_ref[...].astype(jnp.float32),
                       preferred_element_type=jnp.float32)
    ltokb = jnp.minimum(1.0, segb)
    gtok_row = jnp.minimum(1.0, gseg_row)
    gtok_col = jnp.minimum(1.0, gseg_col)

    l2g3 = (ltokb[:, :, None] == gtok_row[None, :, :]).astype(jnp.float32)
    l2g_ref[...] = l2g3.reshape(L, G)
    g2g_ref[...] = (gtok_col == gtok_row).astype(jnp.float32)
    same_ref[...] = (gseg_col == gseg_row).astype(jnp.float32)

    iotg = jax.lax.broadcasted_iota(jnp.int32, (1, 1, G), 2)
    ind3 = (pid_ref[...][:, :, None] == iotg).astype(jnp.float32)
    ind_ref[...] = ind3.reshape(L, G)

    pad = jnp.full((1, R), -1.0, jnp.float32)
    segp = jnp.concatenate([pad, segb, pad], axis=0)
    seg3 = jnp.concatenate([segp[:-2], segp[1:-1], segp[2:]], axis=1)
    qi = jax.lax.broadcasted_iota(jnp.int32, (R, 3 * R), 1)
    pi = jax.lax.broadcasted_iota(jnp.int32, (R, 3 * R), 0)
    win = (jnp.abs(qi - (R + pi)) <= R)
    locm = ((segb[:, :, None] == seg3[:, None, :]) & win[None]).astype(jnp.float32)
    locm_ref[...] = locm.reshape(L, 3 * R)


def _masks(long_bp, glob_bp, pid):
    outs = (
        jax.ShapeDtypeStruct((L, 3 * R), jnp.float32),
        jax.ShapeDtypeStruct((L, G), jnp.float32),
        jax.ShapeDtypeStruct((G, G), jnp.float32),
        jax.ShapeDtypeStruct((L, G), jnp.float32),
        jax.ShapeDtypeStruct((G, G), jnp.float32),
    )
    return pl.pallas_call(_mask_body, out_shape=outs)(
        long_bp.reshape(NB, R), glob_bp.reshape(1, G),
        glob_bp.reshape(G, 1), pid.reshape(NB, R))


def _dotc(a, b):
    return jax.lax.dot_general(a, b, (((1,), (1,)), ((), ())),
                               preferred_element_type=jnp.float32)


def _attn_body(ql_ref, kl_ref, vl_ref, qg_ref, kg_ref, vg_ref,
               locm_ref, l2g_ref, g2l_ref, g2g_ref, ind_ref, indt_ref,
               same_ref, relsel_ref, relgsel_ref, rel3_ref,
               ctxl_ref, ctxg_ref):
    ql = ql_ref[0]
    kl = kl_ref[0]
    vl = vl_ref[0]
    qg = qg_ref[0]
    kg = kg_ref[0]
    vg = vg_ref[0]

    qb = ql.reshape(NB, R, DH)
    kb = kl.reshape(NB, R, DH)
    vb = vl.reshape(NB, R, DH)
    z = jnp.zeros((1, R, DH), jnp.float32)
    kp = jnp.concatenate([z, kb, z], axis=0)
    vp = jnp.concatenate([z, vb, z], axis=0)
    k3 = jnp.concatenate([kp[:-2], kp[1:-1], kp[2:]], axis=1)
    v3 = jnp.concatenate([vp[:-2], vp[1:-1], vp[2:]], axis=1)

    s_loc = jax.lax.dot_general(qb, k3, (((2,), (2,)), ((0,), (0,))),
                                preferred_element_type=jnp.float32)
    s_rel = jax.lax.dot_general(
        qb.transpose(1, 0, 2), relsel_ref[...],
        (((2,), (2,)), ((0,), (0,))),
        preferred_element_type=jnp.float32).transpose(1, 0, 2)
    s_loc = (s_loc + s_rel) * SCALE + (1.0 - locm_ref[...].reshape(NB, R, 3 * R)) * NEG

    crel = _dotc(ql, rel3_ref[...])
    s_l2g = _dotc(ql, kg) + crel[:, 0:1] + ind_ref[...] * (crel[:, 1:2] - crel[:, 0:1])
    s_l2g = s_l2g * SCALE + (1.0 - l2g_ref[...]) * NEG

    s = jnp.concatenate([s_loc.reshape(L, 3 * R), s_l2g], axis=-1)
    m = jnp.max(s, axis=-1, keepdims=True)
    e = jnp.exp(s - m)
    a = e / jnp.sum(e, axis=-1, keepdims=True)
    a_loc = a[:, :3 * R].reshape(NB, R, 3 * R)
    ctx = jax.lax.dot_general(a_loc, v3, (((2,), (1,)), ((0,), (0,))),
                              preferred_element_type=jnp.float32)
    ctxl_ref[0] = ctx.reshape(L, DH) + jnp.dot(
        a[:, 3 * R:], vg, preferred_element_type=jnp.float32)

    crelg = _dotc(qg, rel3_ref[...])
    base = jnp.sum(relgsel_ref[...] * qg[:, None, :], axis=-1)
    s_gg = _dotc(qg, kg) + jnp.where(same_ref[...] > 0.0, base, crelg[:, 2:3])
    s_gg = s_gg * SCALE + (1.0 - g2g_ref[...]) * NEG
    s_gl = _dotc(qg, kl) + crelg[:, 0:1] + indt_ref[...] * (crelg[:, 1:2] - crelg[:, 0:1])
    s_gl = s_gl * SCALE + (1.0 - g2l_ref[...]) * NEG

    sg = jnp.concatenate([s_gg, s_gl], axis=-1)
    mg = jnp.max(sg, axis=-1, keepdims=True)
    eg = jnp.exp(sg - mg)
    ag = eg / jnp.sum(eg, axis=-1, keepdims=True)
    ctxg_ref[0] = (jnp.dot(ag[:, :G], vg, preferred_element_type=jnp.float32)
                   + jnp.dot(ag[:, G:], vl, preferred_element_type=jnp.float32))


def _attention(ql, kl, vl, qg, kg, vg, locm, l2g, g2l, g2g, ind, indt,
               same, relsel, relgsel, rel3):
    full = lambda shape: pl.BlockSpec(shape, lambda i: tuple(0 for _ in shape))
    outs = (
        jax.ShapeDtypeStruct((H, L, DH), jnp.float32),
        jax.ShapeDtypeStruct((H, G, DH), jnp.float32),
    )
    return pl.pallas_call(
        _attn_body,
        grid=(H,),
        in_specs=[
            pl.BlockSpec((1, L, DH), lambda i: (i, 0, 0)),
            pl.BlockSpec((1, L, DH), lambda i: (i, 0, 0)),
            pl.BlockSpec((1, L, DH), lambda i: (i, 0, 0)),
            pl.BlockSpec((1, G, DH), lambda i: (i, 0, 0)),
            pl.BlockSpec((1, G, DH), lambda i: (i, 0, 0)),
            pl.BlockSpec((1, G, DH), lambda i: (i, 0, 0)),
            full((L, 3 * R)),
            full((L, G)),
            full((G, L)),
            full((G, G)),
            full((L, G)),
            full((G, L)),
            full((G, G)),
            full((R, 3 * R, DH)),
            full((G, G, DH)),
            full((3, DH)),
        ],
        out_specs=(
            pl.BlockSpec((1, L, DH), lambda i: (i, 0, 0)),
            pl.BlockSpec((1, G, DH), lambda i: (i, 0, 0)),
        ),
        out_shape=outs,
    )(ql, kl, vl, qg, kg, vg, locm, l2g, g2l, g2g, ind, indt, same,
      relsel, relgsel, rel3)


def kernel(params, long_token_ids, global_token_ids, long_token_type_ids,
           global_token_type_ids, long_breakpoints, long_paragraph_ids,
           global_breakpoints):
    p = params
    ids = jnp.concatenate([long_token_ids[0], global_token_ids[0]])
    type_ids = jnp.concatenate([long_token_type_ids[0], global_token_type_ids[0]])

    rows = _emb_gather(p['emb'], ids)
    x = _embed_ln(rows, type_ids, p['type_emb'], p['emb_ln_s'], p['emb_ln_b'])

    locm, l2g, g2g, ind, same = _masks(
        long_breakpoints[0], global_breakpoints[0], long_paragraph_ids[0])
    g2l = l2g.T
    indt = ind.T

    for lp in p['layers']:
        wqkv = jnp.concatenate([lp['wq'], lp['wk'], lp['wv']], axis=1)
        bqkv = jnp.concatenate([lp['bq'], lp['bk'], lp['bv']])
        qkv = _matmul(x, wqkv, bqkv)

        ql = qkv[:L, 0:D].reshape(L, H, DH).transpose(1, 0, 2)
        kl = qkv[:L, D:2 * D].reshape(L, H, DH).transpose(1, 0, 2)
        vl = qkv[:L, 2 * D:3 * D].reshape(L, H, DH).transpose(1, 0, 2)
        qg = qkv[L:, 0:D].reshape(G, H, DH).transpose(1, 0, 2)
        kg = qkv[L:, D:2 * D].reshape(G, H, DH).transpose(1, 0, 2)
        vg = qkv[L:, 2 * D:3 * D].reshape(G, H, DH).transpose(1, 0, 2)

        rel = lp['rel']
        relsel = rel[jnp.asarray(_L2L_IDS)]
        relgsel = rel[jnp.asarray(_G2G_IDS)]
        rel3 = rel[RELVOC:RELVOC + 3]

        ctxl, ctxg = _attention(ql, kl, vl, qg, kg, vg, locm, l2g, g2l,
                                g2g, ind, indt, same, relsel, relgsel, rel3)
        ctx = jnp.concatenate([
            ctxl.transpose(1, 0, 2).reshape(L, D),
            ctxg.transpose(1, 0, 2).reshape(G, D),
        ], axis=0)
        x = _matmul_ln(ctx, lp['wo'], lp['bo'], x, lp['ln1_s'], lp['ln1_b'])
        h1 = _matmul(x, lp['w1'], lp['b1'], act='gelu', lowp=True)
        x = _matmul_ln(h1, lp['w2'], lp['b2'], x, lp['ln2_s'], lp['ln2_b'],
                       lowp=True)

    wout = jnp.concatenate([p['long_w'], p['glob_w']], axis=1)
    bout = jnp.stack([p['long_b'][0], p['glob_b'][0]]).reshape(1, 2)
    return _head(x, wout, bout)[None]


# speedup vs baseline: 1.0013x; 1.0013x over previous
import functools

import jax
import jax.numpy as jnp
import numpy as np
from jax.experimental import pallas as pl
from jax.experimental.pallas import tpu as pltpu

VOCAB = 30522
TYPES = 16
D = 768
H = 12
DH = 64
L = 2048
G = 128
R = 64
MAXD = 12
FF = 3072
RELVOC = 2 * MAXD + 1
TOTREL = RELVOC + 3
NB = L // R
T = L + G
TM = 136
GROWS = 16
SCALE = 1.0 / np.sqrt(DH)
NEG = -1e9

_pidx = np.arange(R)
_qidx = np.arange(3 * R)
_dloc = _qidx[None, :] - (R + _pidx[:, None])


def _np_rel_id(d):
    return np.where(d >= 0, np.minimum(d, MAXD), MAXD + np.minimum(-d, MAXD))


_L2L_IDS = _np_rel_id(_dloc)
_dg = np.arange(G)[None, :] - np.arange(G)[:, None]
_G2G_IDS = _np_rel_id(_dg)


def _gather_body(ids_ref, *refs):
    del ids_ref
    out_ref = refs[GROWS]
    for j in range(GROWS):
        out_ref[j, :] = refs[j][0, 0, :]


def _emb_gather(emb, ids):
    emb3 = emb.reshape(VOCAB, 1, D)

    def mk(j):
        return pl.BlockSpec((1, 1, D),
                            lambda i, ids_ref, j=j: (ids_ref[i * GROWS + j], 0, 0))

    grid_spec = pltpu.PrefetchScalarGridSpec(
        num_scalar_prefetch=1,
        grid=(T // GROWS,),
        in_specs=[mk(j) for j in range(GROWS)],
        out_specs=pl.BlockSpec((GROWS, D), lambda i, ids_ref: (i, 0)),
    )
    return pl.pallas_call(
        _gather_body,
        grid_spec=grid_spec,
        out_shape=jax.ShapeDtypeStruct((T, D), jnp.float32),
    )(ids, *([emb3] * GROWS))


def _embln_body(x_ref, tids_ref, temb_ref, s_ref, b_ref, o_ref):
    x = x_ref[...]
    tids = tids_ref[0, 0, :]
    oh = (tids[:, None] == jax.lax.broadcasted_iota(jnp.int32, (1, TYPES), 1)
          ).astype(jnp.float32)
    x = x + jnp.dot(oh, temb_ref[...], preferred_element_type=jnp.float32)
    m = jnp.mean(x, axis=-1, keepdims=True)
    v = jnp.mean(jnp.square(x - m), axis=-1, keepdims=True)
    o_ref[...] = (x - m) / jnp.sqrt(v + 1e-6) * s_ref[...] + b_ref[...]


def _embed_ln(rows, type_ids, type_emb, s, b):
    return pl.pallas_call(
        _embln_body,
        grid=(T // TM,),
        in_specs=[
            pl.BlockSpec((TM, D), lambda i: (i, 0)),
            pl.BlockSpec((1, 1, TM), lambda i: (i, 0, 0)),
            pl.BlockSpec((TYPES, D), lambda i: (0, 0)),
            pl.BlockSpec((1, D), lambda i: (0, 0)),
            pl.BlockSpec((1, D), lambda i: (0, 0)),
        ],
        out_specs=pl.BlockSpec((TM, D), lambda i: (i, 0)),
        out_shape=jax.ShapeDtypeStruct((T, D), jnp.float32),
    )(rows, type_ids.reshape(T // TM, 1, TM), type_emb,
      s.reshape(1, D), b.reshape(1, D))


def _mm_body(x_ref, w_ref, b_ref, o_ref, *, act, lowp=False):
    x = x_ref[...]
    w = w_ref[...]
    if lowp:
        x = x.astype(jnp.bfloat16)
        w = w.astype(jnp.bfloat16)
    y = jnp.dot(x, w, preferred_element_type=jnp.float32)
    y = y + b_ref[...]
    if act == 'gelu':
        y = jax.nn.gelu(y)
    elif act == 'sigmoid':
        y = jax.nn.sigmoid(y)
    o_ref[...] = y


def _matmul(x, w, b, act='none', lowp=False):
    M, K = x.shape
    N = w.shape[1]
    return pl.pallas_call(
        functools.partial(_mm_body, act=act, lowp=lowp),
        grid=(M // TM,),
        in_specs=[
            pl.BlockSpec((TM, K), lambda i: (i, 0)),
            pl.BlockSpec((K, N), lambda i: (0, 0)),
            pl.BlockSpec((1, N), lambda i: (0, 0)),
        ],
        out_specs=pl.BlockSpec((TM, N), lambda i: (i, 0)),
        out_shape=jax.ShapeDtypeStruct((M, N), jnp.float32),
    )(x, w, b.reshape(1, N))


def _mm_ln_body(x_ref, w_ref, b_ref, r_ref, s_ref, bb_ref, o_ref, *, lowp=False):
    x = x_ref[...]
    w = w_ref[...]
    if lowp:
        x = x.astype(jnp.bfloat16)
        w = w.astype(jnp.bfloat16)
    y = r_ref[...] + jnp.dot(x, w, preferred_element_type=jnp.float32) + b_ref[...]
    m = jnp.mean(y, axis=-1, keepdims=True)
    v = jnp.mean(jnp.square(y - m), axis=-1, keepdims=True)
    o_ref[...] = (y - m) / jnp.sqrt(v + 1e-6) * s_ref[...] + bb_ref[...]


def _matmul_ln(x, w, b, res, s, bb, lowp=False):
    M, K = x.shape
    N = w.shape[1]
    return pl.pallas_call(
        functools.partial(_mm_ln_body, lowp=lowp),
        grid=(M // TM,),
        in_specs=[
            pl.BlockSpec((TM, K), lambda i: (i, 0)),
            pl.BlockSpec((K, N), lambda i: (0, 0)),
            pl.BlockSpec((1, N), lambda i: (0, 0)),
            pl.BlockSpec((TM, N), lambda i: (i, 0)),
            pl.BlockSpec((1, N), lambda i: (0, 0)),
            pl.BlockSpec((1, N), lambda i: (0, 0)),
        ],
        out_specs=pl.BlockSpec((TM, N), lambda i: (i, 0)),
        out_shape=jax.ShapeDtypeStruct((M, N), jnp.float32),
    )(x, w, b.reshape(1, N), res, s.reshape(1, N), bb.reshape(1, N))


def _head_body(x_ref, w_ref, b_ref, o_ref):
    i = pl.program_id(0)
    y = jnp.dot(x_ref[...], w_ref[...], preferred_element_type=jnp.float32)
    y = y + b_ref[...]
    rows = i * TM + jax.lax.broadcasted_iota(jnp.int32, (TM, 1), 0)
    o_ref[...] = jax.nn.sigmoid(jnp.where(rows >= L, y[:, 1:2], y[:, 0:1]))


def _head(x, w2, b2):
    return pl.pallas_call(
        _head_body,
        grid=(T // TM,),
        in_specs=[
            pl.BlockSpec((TM, D), lambda i: (i, 0)),
            pl.BlockSpec((D, 2), lambda i: (0, 0)),
            pl.BlockSpec((1, 2), lambda i: (0, 0)),
        ],
        out_specs=pl.BlockSpec((TM, 1), lambda i: (i, 0)),
        out_shape=jax.ShapeDtypeStruct((T, 1), jnp.float32),
    )(x, w2, b2)


def _suffix_cumsum(bm):
    P, Q = bm.shape
    jq = jax.lax.broadcasted_iota(jnp.int32, (Q, Q), 0)
    iq = jax.lax.broadcasted_iota(jnp.int32, (Q, Q), 1)
    tq = (jq >= iq).astype(jnp.float32)
    w = jnp.dot(bm, tq, preferred_element_type=jnp.float32)
    rowtot = w[:, 0:1]
    jp = jax.lax.broadcasted_iota(jnp.int32, (P, P), 0)
    ip = jax.lax.broadcasted_iota(jnp.int32, (P, P), 1)
    tp = (jp > ip).astype(jnp.float32)
    tail = jnp.dot(rowtot.reshape(1, P), tp,
                   preferred_element_type=jnp.float32)
    return w + tail.reshape(P, 1)


def _mask_body(lbp_ref, gbp_ref, gbpc_ref, pid_ref,
               locm_ref, l2g_ref, g2g_ref, ind_ref, same_ref):
    segb = _suffix_cumsum(lbp_ref[...].astype(jnp.float32))
    gbpf = gbp_ref[...].astype(jnp.float32)
    jq = jax.lax.broadcasted_iota(jnp.int32, (G, G), 0)
    iq = jax.lax.broadcasted_iota(jnp.int32, (G, G), 1)
    tq = (jq >= iq).astype(jnp.float32)
    gseg_row = jnp.dot(gbpf, tq, preferred_element_type=jnp.float32)
    tq2 = (iq >= jq).astype(jnp.float32)
    gseg_col = jnp.dot(tq2, gbpc_ref[...].astype(jnp.float32),
                       preferred_element_type=jnp.float32)
    ltokb = jnp.minimum(1.0, segb)
    gtok_row = jnp.minimum(1.0, gseg_row)
    gtok_col = jnp.minimum(1.0, gseg_col)

    l2g3 = (ltokb[:, :, None] == gtok_row[None, :, :]).astype(jnp.float32)
    l2g_ref[...] = l2g3.reshape(L, G)
    g2g_ref[...] = (gtok_col == gtok_row).astype(jnp.float32)
    same_ref[...] = (gseg_col == gseg_row).astype(jnp.float32)

    iotg = jax.lax.broadcasted_iota(jnp.int32, (1, 1, G), 2)
    ind3 = (pid_ref[...][:, :, None] == iotg).astype(jnp.float32)
    ind_ref[...] = ind3.reshape(L, G)

    pad = jnp.full((1, R), -1.0, jnp.float32)
    segp = jnp.concatenate([pad, segb, pad], axis=0)
    seg3 = jnp.concatenate([segp[:-2], segp[1:-1], segp[2:]], axis=1)
    qi = jax.lax.broadcasted_iota(jnp.int32, (R, 3 * R), 1)
    pi = jax.lax.broadcasted_iota(jnp.int32, (R, 3 * R), 0)
    win = (jnp.abs(qi - (R + pi)) <= R)
    locm = ((segb[:, :, None] == seg3[:, None, :]) & win[None]).astype(jnp.float32)
    locm_ref[...] = locm.reshape(L, 3 * R)


def _masks(long_bp, glob_bp, pid):
    outs = (
        jax.ShapeDtypeStruct((L, 3 * R), jnp.float32),
        jax.ShapeDtypeStruct((L, G), jnp.float32),
        jax.ShapeDtypeStruct((G, G), jnp.float32),
        jax.ShapeDtypeStruct((L, G), jnp.float32),
        jax.ShapeDtypeStruct((G, G), jnp.float32),
    )
    return pl.pallas_call(_mask_body, out_shape=outs)(
        long_bp.reshape(NB, R), glob_bp.reshape(1, G),
        glob_bp.reshape(G, 1), pid.reshape(NB, R))


def _dotc(a, b):
    return jax.lax.dot_general(a, b, (((1,), (1,)), ((), ())),
                               preferred_element_type=jnp.float32)


def _attn_body(ql_ref, kl_ref, vl_ref, qg_ref, kg_ref, vg_ref,
               locm_ref, l2g_ref, g2l_ref, g2g_ref, ind_ref, indt_ref,
               same_ref, relsel_ref, relgsel_ref, rel3_ref,
               ctxl_ref, ctxg_ref):
    ql = ql_ref[0]
    kl = kl_ref[0]
    vl = vl_ref[0]
    qg = qg_ref[0]
    kg = kg_ref[0]
    vg = vg_ref[0]

    qb = ql.reshape(NB, R, DH)
    kb = kl.reshape(NB, R, DH)
    vb = vl.reshape(NB, R, DH)
    z = jnp.zeros((1, R, DH), jnp.float32)
    kp = jnp.concatenate([z, kb, z], axis=0)
    vp = jnp.concatenate([z, vb, z], axis=0)
    k3 = jnp.concatenate([kp[:-2], kp[1:-1], kp[2:]], axis=1)
    v3 = jnp.concatenate([vp[:-2], vp[1:-1], vp[2:]], axis=1)

    s_loc = jax.lax.dot_general(qb, k3, (((2,), (2,)), ((0,), (0,))),
                                preferred_element_type=jnp.float32)
    s_rel = jax.lax.dot_general(
        qb.transpose(1, 0, 2), relsel_ref[...],
        (((2,), (2,)), ((0,), (0,))),
        preferred_element_type=jnp.float32).transpose(1, 0, 2)
    s_loc = (s_loc + s_rel) * SCALE + (1.0 - locm_ref[...].reshape(NB, R, 3 * R)) * NEG

    crel = _dotc(ql, rel3_ref[...])
    s_l2g = _dotc(ql, kg) + crel[:, 0:1] + ind_ref[...] * (crel[:, 1:2] - crel[:, 0:1])
    s_l2g = s_l2g * SCALE + (1.0 - l2g_ref[...]) * NEG

    s = jnp.concatenate([s_loc.reshape(L, 3 * R), s_l2g], axis=-1)
    m = jnp.max(s, axis=-1, keepdims=True)
    e = jnp.exp(s - m)
    a = e / jnp.sum(e, axis=-1, keepdims=True)
    a_loc = a[:, :3 * R].reshape(NB, R, 3 * R)
    ctx = jax.lax.dot_general(a_loc, v3, (((2,), (1,)), ((0,), (0,))),
                              preferred_element_type=jnp.float32)
    ctxl_ref[0] = ctx.reshape(L, DH) + jnp.dot(
        a[:, 3 * R:], vg, preferred_element_type=jnp.float32)

    crelg = _dotc(qg, rel3_ref[...])
    base = jnp.sum(relgsel_ref[...] * qg[:, None, :], axis=-1)
    s_gg = _dotc(qg, kg) + jnp.where(same_ref[...] > 0.0, base, crelg[:, 2:3])
    s_gg = s_gg * SCALE + (1.0 - g2g_ref[...]) * NEG
    s_gl = _dotc(qg, kl) + crelg[:, 0:1] + indt_ref[...] * (crelg[:, 1:2] - crelg[:, 0:1])
    s_gl = s_gl * SCALE + (1.0 - g2l_ref[...]) * NEG

    sg = jnp.concatenate([s_gg, s_gl], axis=-1)
    mg = jnp.max(sg, axis=-1, keepdims=True)
    eg = jnp.exp(sg - mg)
    ag = eg / jnp.sum(eg, axis=-1, keepdims=True)
    ctxg_ref[0] = (jnp.dot(ag[:, :G], vg, preferred_element_type=jnp.float32)
                   + jnp.dot(ag[:, G:], vl, preferred_element_type=jnp.float32))


def _attention(ql, kl, vl, qg, kg, vg, locm, l2g, g2l, g2g, ind, indt,
               same, relsel, relgsel, rel3):
    full = lambda shape: pl.BlockSpec(shape, lambda i: tuple(0 for _ in shape))
    outs = (
        jax.ShapeDtypeStruct((H, L, DH), jnp.float32),
        jax.ShapeDtypeStruct((H, G, DH), jnp.float32),
    )
    return pl.pallas_call(
        _attn_body,
        grid=(H,),
        in_specs=[
            pl.BlockSpec((1, L, DH), lambda i: (i, 0, 0)),
            pl.BlockSpec((1, L, DH), lambda i: (i, 0, 0)),
            pl.BlockSpec((1, L, DH), lambda i: (i, 0, 0)),
            pl.BlockSpec((1, G, DH), lambda i: (i, 0, 0)),
            pl.BlockSpec((1, G, DH), lambda i: (i, 0, 0)),
            pl.BlockSpec((1, G, DH), lambda i: (i, 0, 0)),
            full((L, 3 * R)),
            full((L, G)),
            full((G, L)),
            full((G, G)),
            full((L, G)),
            full((G, L)),
            full((G, G)),
            full((R, 3 * R, DH)),
            full((G, G, DH)),
            full((3, DH)),
        ],
        out_specs=(
            pl.BlockSpec((1, L, DH), lambda i: (i, 0, 0)),
            pl.BlockSpec((1, G, DH), lambda i: (i, 0, 0)),
        ),
        out_shape=outs,
    )(ql, kl, vl, qg, kg, vg, locm, l2g, g2l, g2g, ind, indt, same,
      relsel, relgsel, rel3)


def kernel(params, long_token_ids, global_token_ids, long_token_type_ids,
           global_token_type_ids, long_breakpoints, long_paragraph_ids,
           global_breakpoints):
    p = params
    ids = jnp.concatenate([long_token_ids[0], global_token_ids[0]])
    type_ids = jnp.concatenate([long_token_type_ids[0], global_token_type_ids[0]])

    rows = _emb_gather(p['emb'], ids)
    x = _embed_ln(rows, type_ids, p['type_emb'], p['emb_ln_s'], p['emb_ln_b'])

    locm, l2g, g2g, ind, same = _masks(
        long_breakpoints[0], global_breakpoints[0], long_paragraph_ids[0])
    g2l = l2g.T
    indt = ind.T

    for lp in p['layers']:
        wqkv = jnp.concatenate([lp['wq'], lp['wk'], lp['wv']], axis=1)
        bqkv = jnp.concatenate([lp['bq'], lp['bk'], lp['bv']])
        qkv = _matmul(x, wqkv, bqkv)

        ql = qkv[:L, 0:D].reshape(L, H, DH).transpose(1, 0, 2)
        kl = qkv[:L, D:2 * D].reshape(L, H, DH).transpose(1, 0, 2)
        vl = qkv[:L, 2 * D:3 * D].reshape(L, H, DH).transpose(1, 0, 2)
        qg = qkv[L:, 0:D].reshape(G, H, DH).transpose(1, 0, 2)
        kg = qkv[L:, D:2 * D].reshape(G, H, DH).transpose(1, 0, 2)
        vg = qkv[L:, 2 * D:3 * D].reshape(G, H, DH).transpose(1, 0, 2)

        rel = lp['rel']
        relsel = rel[jnp.asarray(_L2L_IDS)]
        relgsel = rel[jnp.asarray(_G2G_IDS)]
        rel3 = rel[RELVOC:RELVOC + 3]

        ctxl, ctxg = _attention(ql, kl, vl, qg, kg, vg, locm, l2g, g2l,
                                g2g, ind, indt, same, relsel, relgsel, rel3)
        ctx = jnp.concatenate([
            ctxl.transpose(1, 0, 2).reshape(L, D),
            ctxg.transpose(1, 0, 2).reshape(G, D),
        ], axis=0)
        x = _matmul_ln(ctx, lp['wo'], lp['bo'], x, lp['ln1_s'], lp['ln1_b'])
        h1 = _matmul(x, lp['w1'], lp['b1'], act='gelu')
        x = _matmul_ln(h1, lp['w2'], lp['b2'], x, lp['ln2_s'], lp['ln2_b'])

    wout = jnp.concatenate([p['long_w'], p['glob_w']], axis=1)
    bout = jnp.stack([p['long_b'][0], p['glob_b'][0]]).reshape(1, 2)
    return _head(x, wout, bout)[None]
